# baseline (speedup 1.0000x reference)
import jax
import jax.numpy as jnp
from jax.experimental import pallas as pl
from jax.experimental.pallas import tpu as pltpu

B, N, D = 1, 2048, 768
H, KVH, DH = 12, 12, 64
BLK, STRIDE = 16, 8
SELBLK, NSEL = 16, 4
WIN = 64
NCB = (N - BLK) // STRIDE + 1
NSB = N // SELBLK
SCALE = DH ** -0.5

TQ = 256
QT = N // TQ
TK = 256
KT = N // TK
CW = 3 * 64
WTOT = 3 * D + 256


def _dot(a, b, ta=False, tb=False):
    dn = (((0 if ta else 1,), (1 if tb else 0,)), ((), ()))
    return jax.lax.dot_general(a.astype(jnp.bfloat16), b.astype(jnp.bfloat16),
                               dn, preferred_element_type=jnp.float32)


def _proj_kernel(x_ref, g_ref, w_ref, b_ref, o_ref):
    x = x_ref[...]
    ms = jnp.mean(x * x, axis=-1, keepdims=True)
    xn = x * jax.lax.rsqrt(ms + 1e-6) * g_ref[...]
    o_ref[...] = _dot(xn, w_ref[...]) + b_ref[...]


def _comp_kernel(k_ref, v_ref, wck_ref, wcv_ref, kpe_ref, vpe_ref,
                 mk_ref, mv_ref, ckf_ref, cvf_ref):
    def one(side_ref, w_ref, pe_ref, mem_ref, out_ref):
        a = side_ref[0]
        w = w_ref[0]
        w1 = w[: STRIDE * DH]
        w2 = w[STRIDE * DH:]
        pe = pe_ref[0]
        ash = jnp.concatenate([a[1:], a[:1]], axis=0)
        a1 = a + pe[:, :STRIDE * DH]
        a2 = ash + pe[:, STRIDE * DH:]
        ck = _dot(a1, w1) + _dot(a2, w2)
        out_ref[0, 0:1, :] = mem_ref[0, 0]
        out_ref[0, 1:NCB + 1, :] = ck[:NCB]
    one(k_ref, wck_ref, kpe_ref, mk_ref, ckf_ref)
    one(v_ref, wcv_ref, vpe_ref, mv_ref, cvf_ref)


def _attn_kernel(q_ref, k_ref, v_ref, ckf_ref, cvf_ref,
                 c_ref, f_ref, s_ref, sbuf):
    t = pl.program_id(1)
    q = q_ref[0]
    rows = t * TQ + jax.lax.broadcasted_iota(jnp.int32, (TQ, 1), 0)

    ckf = ckf_ref[0]
    cvf = cvf_ref[0]
    cs = _dot(q, ckf, tb=True) * SCALE
    colc = jax.lax.broadcasted_iota(jnp.int32, (1, NCB + 1), 1)
    vis = (colc == 0) | ((colc - 1) * STRIDE + BLK - 1 <= rows)
    csm = jnp.where(vis, cs, -1e30)
    cm = jnp.max(csm, axis=1, keepdims=True)
    cpu = jnp.exp(csm - cm)
    cl = jnp.sum(cpu, axis=1, keepdims=True)
    cp = cpu / cl
    c_ref[0] = _dot(cpu, cvf) / cl

    pj = jax.lax.broadcasted_iota(jnp.int32, (NCB + 1, NSB), 0)
    ps = jax.lax.broadcasted_iota(jnp.int32, (NCB + 1, NSB), 1)
    pair = ((pj >= 1) & ((pj - 1) // 2 == ps)).astype(jnp.float32)
    imp = jax.lax.dot_general(cp, pair, (((1,), (0,)), ((), ())),
                              precision=jax.lax.Precision.HIGHEST,
                              preferred_element_type=jnp.float32)
    iot = jax.lax.broadcasted_iota(jnp.int32, (TQ, NSB), 1)
    w = imp
    idxs = []
    for _ in range(NSEL):
        mr = jnp.max(w, axis=1, keepdims=True)
        fi = jnp.min(jnp.where(w == mr, iot, NSB), axis=1, keepdims=True)
        idxs.append(fi)
        w = jnp.where(iot == fi, -1.0, w)
    own = rows // SELBLK
    i0, i1, i2, i3 = idxs

    jj = jax.lax.broadcasted_iota(jnp.int32, (1, TK), 1)

    def p1(kt, m):
        kt0 = kt * TK
        ktile = k_ref[0, pl.ds(kt0, TK), :]
        s = _dot(q, ktile, tb=True) * SCALE
        jglob = kt0 + jj
        sblk = jglob // SELBLK
        gate = ((sblk == i0) | (sblk == i1) | (sblk == i2) |
                (sblk == i3) | (sblk == own))
        fm = gate & (rows >= jglob)
        sf = jnp.where(fm, s, -1e30)
        sbuf[:, pl.ds(kt0, TK)] = sf
        return jnp.maximum(m, jnp.max(sf, axis=1, keepdims=True))

    mf = jax.lax.fori_loop(0, t + 1, p1,
                           jnp.full((TQ, 1), -1e30, jnp.float32))

    def p2(kt, l):
        kt0 = kt * TK
        pf = jnp.exp(sbuf[:, pl.ds(kt0, TK)] - mf)
        sbuf[:, pl.ds(kt0, TK)] = pf
        return l + jnp.sum(pf, axis=1, keepdims=True)

    lf = jax.lax.fori_loop(0, t + 1, p2, jnp.zeros((TQ, 1), jnp.float32))

    def p3(kt, acc):
        kt0 = kt * TK
        fp = (sbuf[:, pl.ds(kt0, TK)] / lf).astype(jnp.bfloat16)
        vtile = v_ref[0, pl.ds(kt0, TK), :]
        return acc + _dot(fp, vtile)

    f_ref[0] = jax.lax.fori_loop(0, t + 1, p3,
                                 jnp.zeros((TQ, DH), jnp.float32))

    start = jnp.maximum(t * TQ - 128, 0)
    kwin = k_ref[0, pl.ds(start, TQ + 128), :]
    vwin = v_ref[0, pl.ds(start, TQ + 128), :]
    sw = _dot(q, kwin, tb=True) * SCALE
    jw = start + jax.lax.broadcasted_iota(jnp.int32, (1, TQ + 128), 1)
    wm = (rows >= jw) & (rows - jw < WIN)
    swm = jnp.where(wm, sw, -1e30)
    mw = jnp.max(swm, axis=1, keepdims=True)
    pw = jnp.exp(swm - mw)
    s_ref[0] = _dot(pw, vwin) / jnp.sum(pw, axis=1, keepdims=True)


def _out_kernel(c_ref, f_ref, s_ref, comb_ref, wo_ref, o_ref):
    gc = jax.nn.sigmoid(comb_ref[:, :3 * H])
    acc = jnp.zeros((TQ, D), jnp.float32)
    for h in range(H):
        ah = (gc[:, h:h + 1] * c_ref[h] +
              gc[:, H + h:H + h + 1] * f_ref[h] +
              gc[:, 2 * H + h:2 * H + h + 1] * s_ref[h])
        acc = acc + _dot(ah, wo_ref[h * DH:(h + 1) * DH, :])
    o_ref[...] = acc


def kernel(x, g_norm, Wq, Wk, Wv, k_pe, v_pe, Wck, Wcv, mem_kv, W_comb, b_comb, Wo):
    x2 = x.reshape(N, D)
    wfull = jnp.concatenate(
        [Wq, Wk, Wv, W_comb,
         jnp.zeros((D, WTOT - 3 * D - 3 * H), jnp.float32)], axis=1)
    bfull = jnp.concatenate(
        [jnp.zeros((3 * D,), jnp.float32), b_comb,
         jnp.zeros((WTOT - 3 * D - 3 * H,), jnp.float32)])[None, :]

    qkvc = pl.pallas_call(
        _proj_kernel,
        grid=(QT,),
        in_specs=[
            pl.BlockSpec((TQ, D), lambda t: (t, 0)),
            pl.BlockSpec((1, D), lambda t: (0, 0)),
            pl.BlockSpec((D, WTOT), lambda t: (0, 0)),
            pl.BlockSpec((1, WTOT), lambda t: (0, 0)),
        ],
        out_specs=pl.BlockSpec((TQ, WTOT), lambda t: (t, 0)),
        out_shape=jax.ShapeDtypeStruct((N, WTOT), jnp.float32),
    )(x2, g_norm[None, :], wfull, bfull)

    q = qkvc[:, :D].reshape(N, H, DH).transpose(1, 0, 2)
    k = qkvc[:, D:2 * D].reshape(N, KVH, DH).transpose(1, 0, 2)
    v = qkvc[:, 2 * D:3 * D].reshape(N, KVH, DH).transpose(1, 0, 2)
    comb = qkvc[:, 3 * D:]

    k2 = k.reshape(KVH, N // STRIDE, STRIDE * DH)
    v2 = v.reshape(KVH, N // STRIDE, STRIDE * DH)
    kpe_flat = k_pe.reshape(KVH, 1, BLK * DH)
    vpe_flat = v_pe.reshape(KVH, 1, BLK * DH)

    ckf, cvf = pl.pallas_call(
        _comp_kernel,
        grid=(KVH,),
        in_specs=[
            pl.BlockSpec((1, N // STRIDE, STRIDE * DH), lambda h: (h, 0, 0)),
            pl.BlockSpec((1, N // STRIDE, STRIDE * DH), lambda h: (h, 0, 0)),
            pl.BlockSpec((1, BLK * DH, DH), lambda h: (h, 0, 0)),
            pl.BlockSpec((1, BLK * DH, DH), lambda h: (h, 0, 0)),
            pl.BlockSpec((1, 1, BLK * DH), lambda h: (h, 0, 0)),
            pl.BlockSpec((1, 1, BLK * DH), lambda h: (h, 0, 0)),
            pl.BlockSpec((1, 1, 1, DH), lambda h: (0, h, 0, 0)),
            pl.BlockSpec((1, 1, 1, DH), lambda h: (1, h, 0, 0)),
        ],
        out_specs=[
            pl.BlockSpec((1, NCB + 1, DH), lambda h: (h, 0, 0)),
            pl.BlockSpec((1, NCB + 1, DH), lambda h: (h, 0, 0)),
        ],
        out_shape=[
            jax.ShapeDtypeStruct((KVH, NCB + 1, DH), jnp.float32),
            jax.ShapeDtypeStruct((KVH, NCB + 1, DH), jnp.float32),
        ],
    )(k2, v2, Wck, Wcv, kpe_flat, vpe_flat, mem_kv, mem_kv)

    c_out, f_out, s_out = pl.pallas_call(
        _attn_kernel,
        grid=(H, QT),
        in_specs=[
            pl.BlockSpec((1, TQ, DH), lambda h, t: (h, t, 0)),
            pl.BlockSpec((1, N, DH), lambda h, t: (h, 0, 0)),
            pl.BlockSpec((1, N, DH), lambda h, t: (h, 0, 0)),
            pl.BlockSpec((1, NCB + 1, DH), lambda h, t: (h, 0, 0)),
            pl.BlockSpec((1, NCB + 1, DH), lambda h, t: (h, 0, 0)),
        ],
        out_specs=[
            pl.BlockSpec((1, TQ, DH), lambda h, t: (h, t, 0)),
            pl.BlockSpec((1, TQ, DH), lambda h, t: (h, t, 0)),
            pl.BlockSpec((1, TQ, DH), lambda h, t: (h, t, 0)),
        ],
        out_shape=[
            jax.ShapeDtypeStruct((H, N, DH), jnp.float32),
            jax.ShapeDtypeStruct((H, N, DH), jnp.float32),
            jax.ShapeDtypeStruct((H, N, DH), jnp.float32),
        ],
        scratch_shapes=[pltpu.VMEM((TQ, N), jnp.float32)],
    )(q, k, v, ckf, cvf)

    out = pl.pallas_call(
        _out_kernel,
        grid=(QT,),
        in_specs=[
            pl.BlockSpec((H, TQ, DH), lambda t: (0, t, 0)),
            pl.BlockSpec((H, TQ, DH), lambda t: (0, t, 0)),
            pl.BlockSpec((H, TQ, DH), lambda t: (0, t, 0)),
            pl.BlockSpec((TQ, WTOT - 3 * D), lambda t: (t, 0)),
            pl.BlockSpec((H * DH, D), lambda t: (0, 0)),
            ],
        out_specs=pl.BlockSpec((TQ, D), lambda t: (t, 0)),
        out_shape=jax.ShapeDtypeStruct((N, D), jnp.float32),
    )(c_out, f_out, s_out, comb, Wo)

    return out[None]


# speedup vs baseline: 1.1515x; 1.1515x over previous
import jax
import jax.numpy as jnp
from jax.experimental import pallas as pl
from jax.experimental.pallas import tpu as pltpu

B, N, D = 1, 2048, 768
H, KVH, DH = 12, 12, 64
BLK, STRIDE = 16, 8
SELBLK, NSEL = 16, 4
WIN = 64
NCB = (N - BLK) // STRIDE + 1
NSB = N // SELBLK
SCALE = DH ** -0.5

TQ = 256
QT = N // TQ
TK = 256
KT = N // TK
CW = 3 * 64
WTOT = 3 * D + 256


def _dot(a, b, ta=False, tb=False):
    dn = (((0 if ta else 1,), (1 if tb else 0,)), ((), ()))
    return jax.lax.dot_general(a.astype(jnp.bfloat16), b.astype(jnp.bfloat16),
                               dn, preferred_element_type=jnp.float32)


def _proj_kernel(x_ref, g_ref, w_ref, b_ref, o_ref):
    x = x_ref[...]
    ms = jnp.mean(x * x, axis=-1, keepdims=True)
    xn = x * jax.lax.rsqrt(ms + 1e-6) * g_ref[...]
    o_ref[...] = _dot(xn, w_ref[...]) + b_ref[...]


def _comp_kernel(k_ref, v_ref, wck_ref, wcv_ref, kpe_ref, vpe_ref,
                 mk_ref, mv_ref, ckf_ref, cvf_ref):
    def one(side_ref, w_ref, pe_ref, mem_ref, out_ref):
        a = side_ref[0]
        w = w_ref[0]
        w1 = w[: STRIDE * DH]
        w2 = w[STRIDE * DH:]
        pe = pe_ref[0]
        ash = jnp.concatenate([a[1:], a[:1]], axis=0)
        a1 = a + pe[:, :STRIDE * DH]
        a2 = ash + pe[:, STRIDE * DH:]
        ck = _dot(a1, w1) + _dot(a2, w2)
        out_ref[0, 0:1, :] = mem_ref[0, 0]
        out_ref[0, 1:NCB + 1, :] = ck[:NCB]
    one(k_ref, wck_ref, kpe_ref, mk_ref, ckf_ref)
    one(v_ref, wcv_ref, vpe_ref, mv_ref, cvf_ref)


def _attn_kernel(q_ref, k_ref, v_ref, ckf_ref, cvf_ref,
                 c_ref, f_ref, s_ref):
    t = pl.program_id(1)
    q = q_ref[0]
    rows = t * TQ + jax.lax.broadcasted_iota(jnp.int32, (TQ, 1), 0)

    ckf = ckf_ref[0]
    cvf = cvf_ref[0]
    cs = _dot(q, ckf, tb=True) * SCALE
    colc = jax.lax.broadcasted_iota(jnp.int32, (1, NCB + 1), 1)
    vis = (colc == 0) | ((colc - 1) * STRIDE + BLK - 1 <= rows)
    csm = jnp.where(vis, cs, -1e30)
    cm = jnp.max(csm, axis=1, keepdims=True)
    cpu = jnp.exp(csm - cm)
    cl = jnp.sum(cpu, axis=1, keepdims=True)
    cp = cpu / cl
    c_ref[0] = _dot(cpu, cvf) / cl

    pj = jax.lax.broadcasted_iota(jnp.int32, (NCB + 1, NSB), 0)
    ps = jax.lax.broadcasted_iota(jnp.int32, (NCB + 1, NSB), 1)
    pair = ((pj >= 1) & ((pj - 1) // 2 == ps)).astype(jnp.float32)
    imp = jax.lax.dot_general(cp, pair, (((1,), (0,)), ((), ())),
                              precision=jax.lax.Precision.HIGHEST,
                              preferred_element_type=jnp.float32)
    iot = jax.lax.broadcasted_iota(jnp.int32, (TQ, NSB), 1)
    w = imp
    idxs = []
    for _ in range(NSEL):
        mr = jnp.max(w, axis=1, keepdims=True)
        fi = jnp.min(jnp.where(w == mr, iot, NSB), axis=1, keepdims=True)
        idxs.append(fi)
        w = jnp.where(iot == fi, -1.0, w)
    own = rows // SELBLK
    i0, i1, i2, i3 = idxs

    kf = k_ref[0]
    vf = v_ref[0]
    s = _dot(q, kf, tb=True) * SCALE
    jf = jax.lax.broadcasted_iota(jnp.int32, (1, N), 1)
    sblk = jf // SELBLK
    gate = ((sblk == i0) | (sblk == i1) | (sblk == i2) |
            (sblk == i3) | (sblk == own))
    fm = gate & (rows >= jf)
    sf = jnp.where(fm, s, -1e30)
    mf = jnp.max(sf, axis=1, keepdims=True)
    pf = jnp.exp(sf - mf)
    fp = (pf / jnp.sum(pf, axis=1, keepdims=True)).astype(jnp.bfloat16)
    f_ref[0] = _dot(fp, vf)

    start = jnp.maximum(t * TQ - 128, 0)
    kwin = k_ref[0, pl.ds(start, TQ + 128), :]
    vwin = v_ref[0, pl.ds(start, TQ + 128), :]
    sw = _dot(q, kwin, tb=True) * SCALE
    jw = start + jax.lax.broadcasted_iota(jnp.int32, (1, TQ + 128), 1)
    wm = (rows >= jw) & (rows - jw < WIN)
    swm = jnp.where(wm, sw, -1e30)
    mw = jnp.max(swm, axis=1, keepdims=True)
    pw = jnp.exp(swm - mw)
    s_ref[0] = _dot(pw, vwin) / jnp.sum(pw, axis=1, keepdims=True)


def _out_kernel(c_ref, f_ref, s_ref, comb_ref, wo_ref, o_ref):
    gc = jax.nn.sigmoid(comb_ref[:, :3 * H])
    ahs = []
    for h in range(H):
        ahs.append(gc[:, h:h + 1] * c_ref[h] +
                   gc[:, H + h:H + h + 1] * f_ref[h] +
                   gc[:, 2 * H + h:2 * H + h + 1] * s_ref[h])
    att = jnp.concatenate(ahs, axis=1)
    o_ref[...] = _dot(att, wo_ref[...])


def kernel(x, g_norm, Wq, Wk, Wv, k_pe, v_pe, Wck, Wcv, mem_kv, W_comb, b_comb, Wo):
    x2 = x.reshape(N, D)
    wfull = jnp.concatenate(
        [Wq, Wk, Wv, W_comb,
         jnp.zeros((D, WTOT - 3 * D - 3 * H), jnp.float32)], axis=1)
    bfull = jnp.concatenate(
        [jnp.zeros((3 * D,), jnp.float32), b_comb,
         jnp.zeros((WTOT - 3 * D - 3 * H,), jnp.float32)])[None, :]

    qkvc = pl.pallas_call(
        _proj_kernel,
        grid=(QT,),
        in_specs=[
            pl.BlockSpec((TQ, D), lambda t: (t, 0)),
            pl.BlockSpec((1, D), lambda t: (0, 0)),
            pl.BlockSpec((D, WTOT), lambda t: (0, 0)),
            pl.BlockSpec((1, WTOT), lambda t: (0, 0)),
        ],
        out_specs=pl.BlockSpec((TQ, WTOT), lambda t: (t, 0)),
        out_shape=jax.ShapeDtypeStruct((N, WTOT), jnp.float32),
        compiler_params=pltpu.CompilerParams(
            dimension_semantics=("parallel",)),
    )(x2, g_norm[None, :], wfull, bfull)

    q = qkvc[:, :D].reshape(N, H, DH).transpose(1, 0, 2)
    k = qkvc[:, D:2 * D].reshape(N, KVH, DH).transpose(1, 0, 2)
    v = qkvc[:, 2 * D:3 * D].reshape(N, KVH, DH).transpose(1, 0, 2)
    comb = qkvc[:, 3 * D:]

    k2 = k.reshape(KVH, N // STRIDE, STRIDE * DH)
    v2 = v.reshape(KVH, N // STRIDE, STRIDE * DH)
    kpe_flat = k_pe.reshape(KVH, 1, BLK * DH)
    vpe_flat = v_pe.reshape(KVH, 1, BLK * DH)

    ckf, cvf = pl.pallas_call(
        _comp_kernel,
        grid=(KVH,),
        in_specs=[
            pl.BlockSpec((1, N // STRIDE, STRIDE * DH), lambda h: (h, 0, 0)),
            pl.BlockSpec((1, N // STRIDE, STRIDE * DH), lambda h: (h, 0, 0)),
            pl.BlockSpec((1, BLK * DH, DH), lambda h: (h, 0, 0)),
            pl.BlockSpec((1, BLK * DH, DH), lambda h: (h, 0, 0)),
            pl.BlockSpec((1, 1, BLK * DH), lambda h: (h, 0, 0)),
            pl.BlockSpec((1, 1, BLK * DH), lambda h: (h, 0, 0)),
            pl.BlockSpec((1, 1, 1, DH), lambda h: (0, h, 0, 0)),
            pl.BlockSpec((1, 1, 1, DH), lambda h: (1, h, 0, 0)),
        ],
        out_specs=[
            pl.BlockSpec((1, NCB + 1, DH), lambda h: (h, 0, 0)),
            pl.BlockSpec((1, NCB + 1, DH), lambda h: (h, 0, 0)),
        ],
        out_shape=[
            jax.ShapeDtypeStruct((KVH, NCB + 1, DH), jnp.float32),
            jax.ShapeDtypeStruct((KVH, NCB + 1, DH), jnp.float32),
        ],
    )(k2, v2, Wck, Wcv, kpe_flat, vpe_flat, mem_kv, mem_kv)

    c_out, f_out, s_out = pl.pallas_call(
        _attn_kernel,
        grid=(H, QT),
        in_specs=[
            pl.BlockSpec((1, TQ, DH), lambda h, t: (h, t, 0)),
            pl.BlockSpec((1, N, DH), lambda h, t: (h, 0, 0)),
            pl.BlockSpec((1, N, DH), lambda h, t: (h, 0, 0)),
            pl.BlockSpec((1, NCB + 1, DH), lambda h, t: (h, 0, 0)),
            pl.BlockSpec((1, NCB + 1, DH), lambda h, t: (h, 0, 0)),
        ],
        out_specs=[
            pl.BlockSpec((1, TQ, DH), lambda h, t: (h, t, 0)),
            pl.BlockSpec((1, TQ, DH), lambda h, t: (h, t, 0)),
            pl.BlockSpec((1, TQ, DH), lambda h, t: (h, t, 0)),
        ],
        out_shape=[
            jax.ShapeDtypeStruct((H, N, DH), jnp.float32),
            jax.ShapeDtypeStruct((H, N, DH), jnp.float32),
            jax.ShapeDtypeStruct((H, N, DH), jnp.float32),
        ],
        compiler_params=pltpu.CompilerParams(
            dimension_semantics=("parallel", "parallel")),
    )(q, k, v, ckf, cvf)

    out = pl.pallas_call(
        _out_kernel,
        grid=(QT,),
        in_specs=[
            pl.BlockSpec((H, TQ, DH), lambda t: (0, t, 0)),
            pl.BlockSpec((H, TQ, DH), lambda t: (0, t, 0)),
            pl.BlockSpec((H, TQ, DH), lambda t: (0, t, 0)),
            pl.BlockSpec((TQ, WTOT - 3 * D), lambda t: (t, 0)),
            pl.BlockSpec((H * DH, D), lambda t: (0, 0)),
            ],
        out_specs=pl.BlockSpec((TQ, D), lambda t: (t, 0)),
        out_shape=jax.ShapeDtypeStruct((N, D), jnp.float32),
        compiler_params=pltpu.CompilerParams(
            dimension_semantics=("parallel",)),
    )(c_out, f_out, s_out, comb, Wo)

    return out[None]


# speedup vs baseline: 1.3567x; 1.1783x over previous
import jax
import jax.numpy as jnp
from jax.experimental import pallas as pl
from jax.experimental.pallas import tpu as pltpu

B, N, D = 1, 2048, 768
H, KVH, DH = 12, 12, 64
BLK, STRIDE = 16, 8
SELBLK, NSEL = 16, 4
WIN = 64
NCB = (N - BLK) // STRIDE + 1
NSB = N // SELBLK
SCALE = DH ** -0.5

GBIG = 2.0 ** 100
TQ = 256
QT = N // TQ
TK = 256
KT = N // TK
CW = 3 * 64
WTOT = 3 * D + 256


def _dot(a, b, ta=False, tb=False):
    dn = (((0 if ta else 1,), (1 if tb else 0,)), ((), ()))
    return jax.lax.dot_general(a.astype(jnp.bfloat16), b.astype(jnp.bfloat16),
                               dn, preferred_element_type=jnp.float32)


def _proj_kernel(x_ref, g_ref, w_ref, b_ref, o_ref):
    x = x_ref[...]
    ms = jnp.mean(x * x, axis=-1, keepdims=True)
    xn = x * jax.lax.rsqrt(ms + 1e-6) * g_ref[...]
    o_ref[...] = _dot(xn, w_ref[...]) + b_ref[...]


def _comp_kernel(k_ref, v_ref, wck_ref, wcv_ref, kpe_ref, vpe_ref,
                 mk_ref, mv_ref, ckf_ref, cvf_ref):
    def one(side_ref, w_ref, pe_ref, mem_ref, out_ref):
        a = side_ref[0]
        w = w_ref[0]
        w1 = w[: STRIDE * DH]
        w2 = w[STRIDE * DH:]
        pe = pe_ref[0]
        ash = jnp.concatenate([a[1:], a[:1]], axis=0)
        a1 = a + pe[:, :STRIDE * DH]
        a2 = ash + pe[:, STRIDE * DH:]
        ck = _dot(a1, w1) + _dot(a2, w2)
        out_ref[0, 0:1, :] = mem_ref[0, 0]
        out_ref[0, 1:NCB + 1, :] = ck[:NCB]
    one(k_ref, wck_ref, kpe_ref, mk_ref, ckf_ref)
    one(v_ref, wcv_ref, vpe_ref, mv_ref, cvf_ref)


def _attn_kernel(q_ref, k_ref, v_ref, ckf_ref, cvf_ref, e_ref,
                 c_ref, f_ref, s_ref):
    t = pl.program_id(1)
    q = q_ref[0] * SCALE
    rows = t * TQ + jax.lax.broadcasted_iota(jnp.int32, (TQ, 1), 0)

    ckf = ckf_ref[0]
    cvf = cvf_ref[0]
    cs = _dot(q, ckf, tb=True)
    colc = jax.lax.broadcasted_iota(jnp.int32, (1, NCB + 1), 1)
    vis = (colc == 0) | ((colc - 1) * STRIDE + BLK - 1 <= rows)
    csm = jnp.where(vis, cs, -1e30)
    cm = jnp.max(csm, axis=1, keepdims=True)
    cpu = jnp.exp(csm - cm)
    cl = jnp.sum(cpu, axis=1, keepdims=True)
    cp = cpu / cl
    c_ref[0] = _dot(cpu, cvf) / cl

    pj = jax.lax.broadcasted_iota(jnp.int32, (NCB + 1, NSB), 0)
    ps = jax.lax.broadcasted_iota(jnp.int32, (NCB + 1, NSB), 1)
    pair = ((pj >= 1) & ((pj - 1) // 2 == ps)).astype(jnp.float32)
    imp = jax.lax.dot_general(cp, pair, (((1,), (0,)), ((), ())),
                              precision=jax.lax.Precision.HIGHEST,
                              preferred_element_type=jnp.float32)
    iot = jax.lax.broadcasted_iota(jnp.int32, (TQ, NSB), 1)
    own = rows // SELBLK
    selm = iot == own
    w = imp
    for _ in range(NSEL):
        mr = jnp.max(w, axis=1, keepdims=True)
        fi = jnp.min(jnp.where(w == mr, iot, NSB), axis=1, keepdims=True)
        pick = iot == fi
        selm = selm | pick
        w = jnp.where(pick, -1.0, w)

    kf = k_ref[0]
    vf = v_ref[0]
    s = _dot(q, kf, tb=True)
    jf = jax.lax.broadcasted_iota(jnp.int32, (1, N), 1)
    gbm = _dot(selm.astype(jnp.float32), e_ref[...])
    fm = (gbm > 0.5) & (rows >= jf)
    sf = jnp.where(fm, s, -1e30)
    mf = jnp.max(sf, axis=1, keepdims=True)
    pf = jnp.exp(sf - mf)
    fp = (pf / jnp.sum(pf, axis=1, keepdims=True)).astype(jnp.bfloat16)
    f_ref[0] = _dot(fp, vf)

    start = jnp.maximum(t * TQ - 128, 0)
    kwin = k_ref[0, pl.ds(start, TQ + 128), :]
    vwin = v_ref[0, pl.ds(start, TQ + 128), :]
    sw = _dot(q, kwin, tb=True)
    jw = start + jax.lax.broadcasted_iota(jnp.int32, (1, TQ + 128), 1)
    wm = (rows >= jw) & (rows - jw < WIN)
    swm = jnp.where(wm, sw, -1e30)
    mw = jnp.max(swm, axis=1, keepdims=True)
    pw = jnp.exp(swm - mw)
    s_ref[0] = _dot(pw, vwin) / jnp.sum(pw, axis=1, keepdims=True)


def _out_kernel(c_ref, f_ref, s_ref, comb_ref, wo_ref, o_ref):
    gc = jax.nn.sigmoid(comb_ref[:, :3 * H])
    ahs = []
    for h in range(H):
        ahs.append(gc[:, h:h + 1] * c_ref[h] +
                   gc[:, H + h:H + h + 1] * f_ref[h] +
                   gc[:, 2 * H + h:2 * H + h + 1] * s_ref[h])
    att = jnp.concatenate(ahs, axis=1)
    o_ref[...] = _dot(att, wo_ref[...])


def kernel(x, g_norm, Wq, Wk, Wv, k_pe, v_pe, Wck, Wcv, mem_kv, W_comb, b_comb, Wo):
    x2 = x.reshape(N, D)
    wfull = jnp.concatenate(
        [Wq, Wk, Wv, W_comb,
         jnp.zeros((D, WTOT - 3 * D - 3 * H), jnp.float32)], axis=1)
    bfull = jnp.concatenate(
        [jnp.zeros((3 * D,), jnp.float32), b_comb,
         jnp.zeros((WTOT - 3 * D - 3 * H,), jnp.float32)])[None, :]

    qkvc = pl.pallas_call(
        _proj_kernel,
        grid=(QT,),
        in_specs=[
            pl.BlockSpec((TQ, D), lambda t: (t, 0)),
            pl.BlockSpec((1, D), lambda t: (0, 0)),
            pl.BlockSpec((D, WTOT), lambda t: (0, 0)),
            pl.BlockSpec((1, WTOT), lambda t: (0, 0)),
        ],
        out_specs=pl.BlockSpec((TQ, WTOT), lambda t: (t, 0)),
        out_shape=jax.ShapeDtypeStruct((N, WTOT), jnp.float32),
        compiler_params=pltpu.CompilerParams(
            dimension_semantics=("parallel",)),
    )(x2, g_norm[None, :], wfull, bfull)

    q = qkvc[:, :D].reshape(N, H, DH).transpose(1, 0, 2)
    k = qkvc[:, D:2 * D].reshape(N, KVH, DH).transpose(1, 0, 2)
    v = qkvc[:, 2 * D:3 * D].reshape(N, KVH, DH).transpose(1, 0, 2)
    comb = qkvc[:, 3 * D:]

    k2 = k.reshape(KVH, N // STRIDE, STRIDE * DH)
    v2 = v.reshape(KVH, N // STRIDE, STRIDE * DH)
    kpe_flat = k_pe.reshape(KVH, 1, BLK * DH)
    vpe_flat = v_pe.reshape(KVH, 1, BLK * DH)

    ckf, cvf = pl.pallas_call(
        _comp_kernel,
        grid=(KVH,),
        in_specs=[
            pl.BlockSpec((1, N // STRIDE, STRIDE * DH), lambda h: (h, 0, 0)),
            pl.BlockSpec((1, N // STRIDE, STRIDE * DH), lambda h: (h, 0, 0)),
            pl.BlockSpec((1, BLK * DH, DH), lambda h: (h, 0, 0)),
            pl.BlockSpec((1, BLK * DH, DH), lambda h: (h, 0, 0)),
            pl.BlockSpec((1, 1, BLK * DH), lambda h: (h, 0, 0)),
            pl.BlockSpec((1, 1, BLK * DH), lambda h: (h, 0, 0)),
            pl.BlockSpec((1, 1, 1, DH), lambda h: (0, h, 0, 0)),
            pl.BlockSpec((1, 1, 1, DH), lambda h: (1, h, 0, 0)),
        ],
        out_specs=[
            pl.BlockSpec((1, NCB + 1, DH), lambda h: (h, 0, 0)),
            pl.BlockSpec((1, NCB + 1, DH), lambda h: (h, 0, 0)),
        ],
        out_shape=[
            jax.ShapeDtypeStruct((KVH, NCB + 1, DH), jnp.float32),
            jax.ShapeDtypeStruct((KVH, NCB + 1, DH), jnp.float32),
        ],
    )(k2, v2, Wck, Wcv, kpe_flat, vpe_flat, mem_kv, mem_kv)

    sb = jnp.arange(NSB, dtype=jnp.int32)
    emat = jnp.where(sb[:, None] == (jnp.arange(N, dtype=jnp.int32) // SELBLK)[None, :],
                     1.0, 0.0).astype(jnp.bfloat16)

    c_out, f_out, s_out = pl.pallas_call(
        _attn_kernel,
        grid=(H, QT),
        in_specs=[
            pl.BlockSpec((1, TQ, DH), lambda h, t: (h, t, 0)),
            pl.BlockSpec((1, N, DH), lambda h, t: (h, 0, 0)),
            pl.BlockSpec((1, N, DH), lambda h, t: (h, 0, 0)),
            pl.BlockSpec((1, NCB + 1, DH), lambda h, t: (h, 0, 0)),
            pl.BlockSpec((1, NCB + 1, DH), lambda h, t: (h, 0, 0)),
            pl.BlockSpec((NSB, N), lambda h, t: (0, 0)),
        ],
        out_specs=[
            pl.BlockSpec((1, TQ, DH), lambda h, t: (h, t, 0)),
            pl.BlockSpec((1, TQ, DH), lambda h, t: (h, t, 0)),
            pl.BlockSpec((1, TQ, DH), lambda h, t: (h, t, 0)),
        ],
        out_shape=[
            jax.ShapeDtypeStruct((H, N, DH), jnp.float32),
            jax.ShapeDtypeStruct((H, N, DH), jnp.float32),
            jax.ShapeDtypeStruct((H, N, DH), jnp.float32),
        ],
        compiler_params=pltpu.CompilerParams(
            dimension_semantics=("parallel", "parallel")),
    )(q, k, v, ckf, cvf, emat)

    out = pl.pallas_call(
        _out_kernel,
        grid=(QT,),
        in_specs=[
            pl.BlockSpec((H, TQ, DH), lambda t: (0, t, 0)),
            pl.BlockSpec((H, TQ, DH), lambda t: (0, t, 0)),
            pl.BlockSpec((H, TQ, DH), lambda t: (0, t, 0)),
            pl.BlockSpec((TQ, WTOT - 3 * D), lambda t: (t, 0)),
            pl.BlockSpec((H * DH, D), lambda t: (0, 0)),
            ],
        out_specs=pl.BlockSpec((TQ, D), lambda t: (t, 0)),
        out_shape=jax.ShapeDtypeStruct((N, D), jnp.float32),
        compiler_params=pltpu.CompilerParams(
            dimension_semantics=("parallel",)),
    )(c_out, f_out, s_out, comb, Wo)

    return out[None]
